# initial kernel scaffold (unmeasured)
import sys

import jax
import jax.numpy as jnp
from jax import lax
from jax.experimental import pallas as pl
from jax.experimental.pallas import tpu as pltpu

N_DEV = 8

try:
    _ds = jax.devices()
    print(
        f"[kernel-debug] n_devices={len(_ds)} kind={_ds[0].device_kind} "
        + " ".join(
            f"{d.id}:{getattr(d, 'coords', None)}/c{getattr(d, 'core_on_chip', None)}"
            for d in _ds
        ),
        file=sys.stderr,
    )
except Exception as _e:
    print(f"[kernel-debug] device probe failed: {_e}", file=sys.stderr)


def kernel(x, w_mat):
    m_per, k = x.shape
    _, n_per = w_mat.shape

    def body(
        x_ref,
        w_ref,
        out_ref,
        comm_ref,
        send_sems,
        recv_sems,
        amax_smem,
        amax_src,
        amax_dst,
        amax_send_sems,
        amax_recv_sems,
    ):
        my = lax.axis_index("i")
        left = lax.rem(my - 1 + N_DEV, N_DEV)
        right = lax.rem(my + 1, N_DEV)

        barrier_sem = pltpu.get_barrier_semaphore()
        for nbr in (left, right):
            pl.semaphore_signal(
                barrier_sem,
                inc=1,
                device_id=(nbr,),
                device_id_type=pl.DeviceIdType.MESH,
            )
        pl.semaphore_wait(barrier_sem, 2)

        def block(origin, chunk):
            y = jnp.maximum(
                jnp.dot(chunk, w_ref[:, :], preferred_element_type=jnp.float32),
                0.0,
            )
            out_ref[pl.ds(origin * m_per, m_per), :] = y
            amax_smem[0] = jnp.maximum(amax_smem[0], jnp.max(y))

        amax_smem[0] = 0.0
        comm_ref[0] = x_ref[:, :]
        block(my, x_ref[:, :])

        for h in range(N_DEV - 1):
            s_slot = h % 2
            r_slot = (h + 1) % 2
            rdma = pltpu.make_async_remote_copy(
                src_ref=comm_ref.at[s_slot],
                dst_ref=comm_ref.at[r_slot],
                send_sem=send_sems.at[h],
                recv_sem=recv_sems.at[h],
                device_id=(right,),
                device_id_type=pl.DeviceIdType.MESH,
            )
            rdma.start()
            rdma.wait()
            origin = lax.rem(my - (h + 1) + N_DEV, N_DEV)
            block(origin, comm_ref[r_slot])

        amax_val = amax_smem[0]
        amax_src[0, :] = jnp.full((128,), amax_val, jnp.float32)
        amax_dst[pl.ds(my, 1), :] = jnp.full((1, 128), amax_val, jnp.float32)

        sends = []
        for off in range(1, N_DEV):
            peer = lax.rem(my + off, N_DEV)
            rd = pltpu.make_async_remote_copy(
                src_ref=amax_src,
                dst_ref=amax_dst.at[pl.ds(my, 1)],
                send_sem=amax_send_sems.at[off - 1],
                recv_sem=amax_recv_sems.at[off - 1],
                device_id=(peer,),
                device_id_type=pl.DeviceIdType.MESH,
            )
            rd.start()
            sends.append(rd)
        for off in range(1, N_DEV):
            src_dev = lax.rem(my - off + N_DEV, N_DEV)
            rd = pltpu.make_async_remote_copy(
                src_ref=amax_src,
                dst_ref=amax_dst.at[pl.ds(src_dev, 1)],
                send_sem=amax_send_sems.at[off - 1],
                recv_sem=amax_recv_sems.at[off - 1],
                device_id=(src_dev,),
                device_id_type=pl.DeviceIdType.MESH,
            )
            rd.wait_recv()
        for rd in sends:
            rd.wait_send()

        gmax = jnp.max(amax_dst[:, :])
        scale = gmax / 448.0
        inv = 1.0 / scale
        y = out_ref[:, :]
        q = jnp.minimum(y * inv, 448.0)
        q = q.astype(jnp.float8_e4m3fn).astype(jnp.float32)
        out_ref[:, :] = q * scale

    return pl.pallas_call(
        body,
        out_shape=jax.ShapeDtypeStruct((N_DEV * m_per, n_per), jnp.float32),
        in_specs=[
            pl.BlockSpec(memory_space=pltpu.VMEM),
            pl.BlockSpec(memory_space=pltpu.VMEM),
        ],
        out_specs=pl.BlockSpec(memory_space=pltpu.VMEM),
        scratch_shapes=[
            pltpu.VMEM((2, m_per, k), jnp.float32),
            pltpu.SemaphoreType.DMA((N_DEV - 1,)),
            pltpu.SemaphoreType.DMA((N_DEV - 1,)),
            pltpu.SMEM((1,), jnp.float32),
            pltpu.VMEM((1, 128), jnp.float32),
            pltpu.VMEM((N_DEV, 128), jnp.float32),
            pltpu.SemaphoreType.DMA((N_DEV - 1,)),
            pltpu.SemaphoreType.DMA((N_DEV - 1,)),
        ],
        compiler_params=pltpu.CompilerParams(collective_id=0),
    )(x, w_mat)


# baseline (device time: 726041 ns/iter reference)
import sys

import jax
import jax.numpy as jnp
from jax import lax
from jax.experimental import pallas as pl
from jax.experimental.pallas import tpu as pltpu

N_DEV = 8

try:
    _ds = jax.devices()
    print(
        f"[kernel-debug] n_devices={len(_ds)} kind={_ds[0].device_kind} "
        + " ".join(
            f"{d.id}:{getattr(d, 'coords', None)}/c{getattr(d, 'core_on_chip', None)}"
            for d in _ds
        ),
        file=sys.stderr,
    )
except Exception as _e:
    print(f"[kernel-debug] device probe failed: {_e}", file=sys.stderr)


def kernel(x, w_mat):
    m_per, k = x.shape
    _, n_per = w_mat.shape

    def body(
        x_ref,
        w_ref,
        out_ref,
        comm_ref,
        send_sems,
        recv_sems,
        amax_smem,
        amax_src,
        amax_dst,
        amax_send_sems,
        amax_recv_sems,
        local_sem,
    ):
        my = lax.axis_index("i")
        left = lax.rem(my - 1 + N_DEV, N_DEV)
        right = lax.rem(my + 1, N_DEV)

        barrier_sem = pltpu.get_barrier_semaphore()
        for nbr in (left, right):
            pl.semaphore_signal(
                barrier_sem,
                inc=1,
                device_id=(nbr,),
                device_id_type=pl.DeviceIdType.MESH,
            )
        pl.semaphore_wait(barrier_sem, 2)

        def block(origin, chunk):
            y = jnp.maximum(
                jnp.dot(chunk, w_ref[:, :], preferred_element_type=jnp.float32),
                0.0,
            )
            out_ref[pl.ds(origin * m_per, m_per), :] = y
            amax_smem[0] = jnp.maximum(amax_smem[0], jnp.max(y))

        amax_smem[0] = 0.0
        cp = pltpu.make_async_copy(x_ref, comm_ref.at[0], local_sem)
        cp.start()
        cp.wait()
        block(my, comm_ref[0])

        for h in range(N_DEV - 1):
            s_slot = h % 2
            r_slot = (h + 1) % 2
            rdma = pltpu.make_async_remote_copy(
                src_ref=comm_ref.at[s_slot],
                dst_ref=comm_ref.at[r_slot],
                send_sem=send_sems.at[h],
                recv_sem=recv_sems.at[h],
                device_id=(right,),
                device_id_type=pl.DeviceIdType.MESH,
            )
            rdma.start()
            rdma.wait()
            origin = lax.rem(my - (h + 1) + N_DEV, N_DEV)
            block(origin, comm_ref[r_slot])

        amax_val = amax_smem[0]
        amax_src[0, :] = jnp.full((128,), amax_val, jnp.float32)
        amax_dst[pl.ds(my, 1), :] = jnp.full((1, 128), amax_val, jnp.float32)

        sends = []
        for off in range(1, N_DEV):
            peer = lax.rem(my + off, N_DEV)
            rd = pltpu.make_async_remote_copy(
                src_ref=amax_src,
                dst_ref=amax_dst.at[pl.ds(my, 1)],
                send_sem=amax_send_sems.at[off - 1],
                recv_sem=amax_recv_sems.at[off - 1],
                device_id=(peer,),
                device_id_type=pl.DeviceIdType.MESH,
            )
            rd.start()
            sends.append(rd)
        for off in range(1, N_DEV):
            src_dev = lax.rem(my - off + N_DEV, N_DEV)
            rd = pltpu.make_async_remote_copy(
                src_ref=amax_src,
                dst_ref=amax_dst.at[pl.ds(src_dev, 1)],
                send_sem=amax_send_sems.at[off - 1],
                recv_sem=amax_recv_sems.at[off - 1],
                device_id=(src_dev,),
                device_id_type=pl.DeviceIdType.MESH,
            )
            rd.wait_recv()
        for rd in sends:
            rd.wait_send()

        gmax = jnp.max(amax_dst[:, :])
        scale = gmax / 448.0
        inv = 1.0 / scale
        for b in range(N_DEV):
            rows = pl.ds(b * m_per, m_per)
            q = jnp.minimum(out_ref[rows, :] * inv, 448.0)
            q = q.astype(jnp.float8_e4m3fn).astype(jnp.float32)
            out_ref[rows, :] = q * scale

    return pl.pallas_call(
        body,
        out_shape=jax.ShapeDtypeStruct((N_DEV * m_per, n_per), jnp.float32),
        in_specs=[
            pl.BlockSpec(memory_space=pl.ANY),
            pl.BlockSpec(memory_space=pltpu.VMEM),
        ],
        out_specs=pl.BlockSpec(memory_space=pltpu.VMEM),
        scratch_shapes=[
            pltpu.VMEM((2, m_per, k), jnp.float32),
            pltpu.SemaphoreType.DMA((N_DEV - 1,)),
            pltpu.SemaphoreType.DMA((N_DEV - 1,)),
            pltpu.SMEM((1,), jnp.float32),
            pltpu.VMEM((1, 128), jnp.float32),
            pltpu.VMEM((N_DEV, 128), jnp.float32),
            pltpu.SemaphoreType.DMA((N_DEV - 1,)),
            pltpu.SemaphoreType.DMA((N_DEV - 1,)),
            pltpu.SemaphoreType.DMA,
        ],
        compiler_params=pltpu.CompilerParams(
            collective_id=0, vmem_limit_bytes=100 * 1024 * 1024
        ),
    )(x, w_mat)


# device time: 370437 ns/iter; 1.9600x vs baseline; 1.9600x over previous
import jax
import jax.numpy as jnp
from jax import lax
from jax.experimental import pallas as pl
from jax.experimental.pallas import tpu as pltpu

N_DEV = 8


def _ring_to_logical(r):
    z = (r >= 4).astype(jnp.int32)
    w = jnp.where(z == 0, r, 7 - r)
    return z * 4 + w


def kernel(x, w_mat):
    m_per, k = x.shape
    _, n_per = w_mat.shape
    m_half = m_per // 2

    def body(
        x_ref,
        w_ref,
        out_ref,
        cw_ref,
        ccw_ref,
        cw_send_sems,
        cw_recv_sems,
        ccw_send_sems,
        ccw_recv_sems,
        credit_cw,
        credit_ccw,
        amax_smem,
        amax_src,
        amax_dst,
        amax_send_sems,
        amax_recv_sems,
        local_sems,
    ):
        my = lax.axis_index("i")
        my_ring = _ring_to_logical(my)
        right_l = _ring_to_logical(lax.rem(my_ring + 1, N_DEV))
        left_l = _ring_to_logical(lax.rem(my_ring + N_DEV - 1, N_DEV))

        barrier_sem = pltpu.get_barrier_semaphore()
        for nbr in (left_l, right_l):
            pl.semaphore_signal(
                barrier_sem,
                inc=1,
                device_id=(nbr,),
                device_id_type=pl.DeviceIdType.MESH,
            )
        pl.semaphore_wait(barrier_sem, 2)

        def gemm_half(origin_l, half, chunk):
            y = jnp.maximum(
                jnp.dot(chunk, w_ref[:, :], preferred_element_type=jnp.float32),
                0.0,
            )
            out_ref[pl.ds(origin_l * m_per + half * m_half, m_half), :] = y
            amax_smem[0] = jnp.maximum(amax_smem[0], jnp.max(y))

        amax_smem[0] = 0.0
        cp_top = pltpu.make_async_copy(
            x_ref.at[pl.ds(0, m_half)], cw_ref.at[0], local_sems.at[0]
        )
        cp_bot = pltpu.make_async_copy(
            x_ref.at[pl.ds(m_half, m_half)], ccw_ref.at[0], local_sems.at[1]
        )
        cp_top.start()
        cp_bot.start()
        cp_top.wait()
        cp_bot.wait()

        for h in range(N_DEV - 1):
            s_slot = h % 2
            r_slot = (h + 1) % 2
            if h >= 2:
                pl.semaphore_wait(credit_cw, 1)
                pl.semaphore_wait(credit_ccw, 1)
            cw = pltpu.make_async_remote_copy(
                src_ref=cw_ref.at[s_slot],
                dst_ref=cw_ref.at[r_slot],
                send_sem=cw_send_sems.at[h],
                recv_sem=cw_recv_sems.at[h],
                device_id=(right_l,),
                device_id_type=pl.DeviceIdType.MESH,
            )
            ccw = pltpu.make_async_remote_copy(
                src_ref=ccw_ref.at[s_slot],
                dst_ref=ccw_ref.at[r_slot],
                send_sem=ccw_send_sems.at[h],
                recv_sem=ccw_recv_sems.at[h],
                device_id=(left_l,),
                device_id_type=pl.DeviceIdType.MESH,
            )
            cw.start()
            ccw.start()

            if h == 0:
                gemm_half(my, 0, cw_ref[0])
                gemm_half(my, 1, ccw_ref[0])
            else:
                cw_origin = _ring_to_logical(
                    lax.rem(my_ring + N_DEV - h, N_DEV)
                )
                ccw_origin = _ring_to_logical(lax.rem(my_ring + h, N_DEV))
                gemm_half(cw_origin, 0, cw_ref[s_slot])
                gemm_half(ccw_origin, 1, ccw_ref[s_slot])

            cw.wait_send()
            ccw.wait_send()
            if 1 <= h <= 5:
                pl.semaphore_signal(
                    credit_cw,
                    inc=1,
                    device_id=(left_l,),
                    device_id_type=pl.DeviceIdType.MESH,
                )
                pl.semaphore_signal(
                    credit_ccw,
                    inc=1,
                    device_id=(right_l,),
                    device_id_type=pl.DeviceIdType.MESH,
                )
            cw.wait_recv()
            ccw.wait_recv()

        gemm_half(_ring_to_logical(lax.rem(my_ring + 1, N_DEV)), 0, cw_ref[1])
        gemm_half(
            _ring_to_logical(lax.rem(my_ring + N_DEV - 1, N_DEV)),
            1,
            ccw_ref[1],
        )

        amax_val = amax_smem[0]
        amax_src[0, :] = jnp.full((128,), amax_val, jnp.float32)
        amax_dst[pl.ds(my, 1), :] = jnp.full((1, 128), amax_val, jnp.float32)

        sends = []
        for off in range(1, N_DEV):
            peer = lax.rem(my + off, N_DEV)
            rd = pltpu.make_async_remote_copy(
                src_ref=amax_src,
                dst_ref=amax_dst.at[pl.ds(my, 1)],
                send_sem=amax_send_sems.at[off - 1],
                recv_sem=amax_recv_sems.at[off - 1],
                device_id=(peer,),
                device_id_type=pl.DeviceIdType.MESH,
            )
            rd.start()
            sends.append(rd)
        for off in range(1, N_DEV):
            src_dev = lax.rem(my - off + N_DEV, N_DEV)
            rd = pltpu.make_async_remote_copy(
                src_ref=amax_src,
                dst_ref=amax_dst.at[pl.ds(src_dev, 1)],
                send_sem=amax_send_sems.at[off - 1],
                recv_sem=amax_recv_sems.at[off - 1],
                device_id=(src_dev,),
                device_id_type=pl.DeviceIdType.MESH,
            )
            rd.wait_recv()
        for rd in sends:
            rd.wait_send()

        gmax = jnp.max(amax_dst[:, :])
        scale = gmax / 448.0
        inv = 1.0 / scale
        for b in range(N_DEV):
            rows = pl.ds(b * m_per, m_per)
            q = jnp.minimum(out_ref[rows, :] * inv, 448.0)
            q = q.astype(jnp.float8_e4m3fn).astype(jnp.float32)
            out_ref[rows, :] = q * scale

    return pl.pallas_call(
        body,
        out_shape=jax.ShapeDtypeStruct((N_DEV * m_per, n_per), jnp.float32),
        in_specs=[
            pl.BlockSpec(memory_space=pl.ANY),
            pl.BlockSpec(memory_space=pltpu.VMEM),
        ],
        out_specs=pl.BlockSpec(memory_space=pltpu.VMEM),
        scratch_shapes=[
            pltpu.VMEM((2, m_half, k), jnp.float32),
            pltpu.VMEM((2, m_half, k), jnp.float32),
            pltpu.SemaphoreType.DMA((N_DEV - 1,)),
            pltpu.SemaphoreType.DMA((N_DEV - 1,)),
            pltpu.SemaphoreType.DMA((N_DEV - 1,)),
            pltpu.SemaphoreType.DMA((N_DEV - 1,)),
            pltpu.SemaphoreType.REGULAR,
            pltpu.SemaphoreType.REGULAR,
            pltpu.SMEM((1,), jnp.float32),
            pltpu.VMEM((1, 128), jnp.float32),
            pltpu.VMEM((N_DEV, 128), jnp.float32),
            pltpu.SemaphoreType.DMA((N_DEV - 1,)),
            pltpu.SemaphoreType.DMA((N_DEV - 1,)),
            pltpu.SemaphoreType.DMA((2,)),
        ],
        compiler_params=pltpu.CompilerParams(
            collective_id=0, vmem_limit_bytes=100 * 1024 * 1024
        ),
    )(x, w_mat)


# device time: 212403 ns/iter; 3.4182x vs baseline; 1.7440x over previous
import jax
import jax.numpy as jnp
from jax import lax
from jax.experimental import pallas as pl
from jax.experimental.pallas import tpu as pltpu

N_DEV = 8


def _ring_to_logical(r):
    z = (r >= 4).astype(jnp.int32)
    w = jnp.where(z == 0, r, 7 - r)
    return z * 4 + w


def kernel(x, w_mat):
    m_per, k = x.shape
    _, n_per = w_mat.shape
    m_half = m_per // 2

    def body(
        x_ref,
        w_ref,
        out_ref,
        cw_ref,
        ccw_ref,
        cw_send_sems,
        cw_recv_sems,
        ccw_send_sems,
        ccw_recv_sems,
        credit_cw,
        credit_ccw,
        amax_smem,
        amax_src,
        amax_dst,
        amax_send_sems,
        amax_recv_sems,
    ):
        my = lax.axis_index("i")
        my_ring = _ring_to_logical(my)
        right_l = _ring_to_logical(lax.rem(my_ring + 1, N_DEV))
        left_l = _ring_to_logical(lax.rem(my_ring + N_DEV - 1, N_DEV))

        barrier_sem = pltpu.get_barrier_semaphore()
        for nbr in (left_l, right_l):
            pl.semaphore_signal(
                barrier_sem,
                inc=1,
                device_id=(nbr,),
                device_id_type=pl.DeviceIdType.MESH,
            )
        pl.semaphore_wait(barrier_sem, 2)

        def gemm_half(origin_l, half, chunk):
            y = jnp.maximum(
                jnp.dot(
                    chunk.astype(jnp.float32),
                    w_ref[:, :],
                    preferred_element_type=jnp.float32,
                ),
                0.0,
            )
            out_ref[pl.ds(origin_l * m_per + half * m_half, m_half), :] = y
            amax_smem[0] = jnp.maximum(amax_smem[0], jnp.max(y))

        amax_smem[0] = 0.0
        cw_ref[0] = x_ref[pl.ds(0, m_half), :].astype(jnp.bfloat16)
        ccw_ref[0] = x_ref[pl.ds(m_half, m_half), :].astype(jnp.bfloat16)

        for h in range(N_DEV - 1):
            s_slot = h % 2
            r_slot = (h + 1) % 2
            if h >= 2:
                pl.semaphore_wait(credit_cw, 1)
                pl.semaphore_wait(credit_ccw, 1)
            cw = pltpu.make_async_remote_copy(
                src_ref=cw_ref.at[s_slot],
                dst_ref=cw_ref.at[r_slot],
                send_sem=cw_send_sems.at[h],
                recv_sem=cw_recv_sems.at[h],
                device_id=(right_l,),
                device_id_type=pl.DeviceIdType.MESH,
            )
            ccw = pltpu.make_async_remote_copy(
                src_ref=ccw_ref.at[s_slot],
                dst_ref=ccw_ref.at[r_slot],
                send_sem=ccw_send_sems.at[h],
                recv_sem=ccw_recv_sems.at[h],
                device_id=(left_l,),
                device_id_type=pl.DeviceIdType.MESH,
            )
            cw.start()
            ccw.start()

            if h == 0:
                gemm_half(my, 0, cw_ref[0])
                gemm_half(my, 1, ccw_ref[0])
            else:
                cw_origin = _ring_to_logical(
                    lax.rem(my_ring + N_DEV - h, N_DEV)
                )
                ccw_origin = _ring_to_logical(lax.rem(my_ring + h, N_DEV))
                gemm_half(cw_origin, 0, cw_ref[s_slot])
                gemm_half(ccw_origin, 1, ccw_ref[s_slot])

            cw.wait_send()
            ccw.wait_send()
            if 1 <= h <= 5:
                pl.semaphore_signal(
                    credit_cw,
                    inc=1,
                    device_id=(left_l,),
                    device_id_type=pl.DeviceIdType.MESH,
                )
                pl.semaphore_signal(
                    credit_ccw,
                    inc=1,
                    device_id=(right_l,),
                    device_id_type=pl.DeviceIdType.MESH,
                )
            cw.wait_recv()
            ccw.wait_recv()

        gemm_half(_ring_to_logical(lax.rem(my_ring + 1, N_DEV)), 0, cw_ref[1])
        gemm_half(
            _ring_to_logical(lax.rem(my_ring + N_DEV - 1, N_DEV)),
            1,
            ccw_ref[1],
        )

        amax_val = amax_smem[0]
        amax_src[0, :] = jnp.full((128,), amax_val, jnp.float32)
        amax_dst[pl.ds(my, 1), :] = jnp.full((1, 128), amax_val, jnp.float32)

        sends = []
        for off in range(1, N_DEV):
            peer = lax.rem(my + off, N_DEV)
            rd = pltpu.make_async_remote_copy(
                src_ref=amax_src,
                dst_ref=amax_dst.at[pl.ds(my, 1)],
                send_sem=amax_send_sems.at[off - 1],
                recv_sem=amax_recv_sems.at[off - 1],
                device_id=(peer,),
                device_id_type=pl.DeviceIdType.MESH,
            )
            rd.start()
            sends.append(rd)
        for off in range(1, N_DEV):
            src_dev = lax.rem(my - off + N_DEV, N_DEV)
            rd = pltpu.make_async_remote_copy(
                src_ref=amax_src,
                dst_ref=amax_dst.at[pl.ds(src_dev, 1)],
                send_sem=amax_send_sems.at[off - 1],
                recv_sem=amax_recv_sems.at[off - 1],
                device_id=(src_dev,),
                device_id_type=pl.DeviceIdType.MESH,
            )
            rd.wait_recv()
        for rd in sends:
            rd.wait_send()

        gmax = jnp.max(amax_dst[:, :])
        scale = gmax / 448.0
        inv = 1.0 / scale
        for b in range(N_DEV):
            rows = pl.ds(b * m_per, m_per)
            q = jnp.minimum(out_ref[rows, :] * inv, 448.0)
            q = q.astype(jnp.float8_e4m3fn).astype(jnp.float32)
            out_ref[rows, :] = q * scale

    return pl.pallas_call(
        body,
        out_shape=jax.ShapeDtypeStruct((N_DEV * m_per, n_per), jnp.float32),
        in_specs=[
            pl.BlockSpec(memory_space=pltpu.VMEM),
            pl.BlockSpec(memory_space=pltpu.VMEM),
        ],
        out_specs=pl.BlockSpec(memory_space=pltpu.VMEM),
        scratch_shapes=[
            pltpu.VMEM((2, m_half, k), jnp.bfloat16),
            pltpu.VMEM((2, m_half, k), jnp.bfloat16),
            pltpu.SemaphoreType.DMA((N_DEV - 1,)),
            pltpu.SemaphoreType.DMA((N_DEV - 1,)),
            pltpu.SemaphoreType.DMA((N_DEV - 1,)),
            pltpu.SemaphoreType.DMA((N_DEV - 1,)),
            pltpu.SemaphoreType.REGULAR,
            pltpu.SemaphoreType.REGULAR,
            pltpu.SMEM((1,), jnp.float32),
            pltpu.VMEM((1, 128), jnp.float32),
            pltpu.VMEM((N_DEV, 128), jnp.float32),
            pltpu.SemaphoreType.DMA((N_DEV - 1,)),
            pltpu.SemaphoreType.DMA((N_DEV - 1,)),
        ],
        compiler_params=pltpu.CompilerParams(
            collective_id=0, vmem_limit_bytes=100 * 1024 * 1024
        ),
    )(x, w_mat)
